# baseline (speedup 1.0000x reference)
import jax
import jax.numpy as jnp
from jax.experimental import pallas as pl


def _mm_body(x_ref, w_ref, o_ref):
    o_ref[...] = jnp.dot(x_ref[...], w_ref[...], preferred_element_type=jnp.float32)


def _mm(x, w, block=1000):
    N, K = x.shape
    _, M = w.shape
    return pl.pallas_call(
        _mm_body,
        grid=(N // block,),
        in_specs=[
            pl.BlockSpec((block, K), lambda i: (i, 0)),
            pl.BlockSpec((K, M), lambda i: (0, 0)),
        ],
        out_specs=pl.BlockSpec((block, M), lambda i: (i, 0)),
        out_shape=jax.ShapeDtypeStruct((N, M), jnp.float32),
    )(x, w)


def _gat_conv(x, src, dst, ew, W, a_s, a_d, b):
    N = x.shape[0]
    Hh, C = a_s.shape
    h = _mm(x, W).reshape(N, Hh, C)
    alpha_src = jnp.sum(h * a_s[None], axis=-1)
    alpha_dst = jnp.sum(h * a_d[None], axis=-1)
    e = jax.nn.leaky_relu(alpha_src[src] + alpha_dst[dst], 0.2)
    emax = jax.ops.segment_max(e, dst, num_segments=N)
    emax = jnp.where(jnp.isfinite(emax), emax, 0.0)
    ex = jnp.exp(e - emax[dst])
    denom = jax.ops.segment_sum(ex, dst, num_segments=N)
    attn = ex / (denom[dst] + 1e-16)
    msg = h[src] * attn[:, :, None] * ew[:, None, None]
    out = jax.ops.segment_sum(msg, dst, num_segments=N)
    return out.reshape(N, Hh * C) + b


def kernel(label, edge_index, weight, request_quantity, emb, Wr, br,
           W1, a_s1, a_d1, b1, W2, a_s2, a_d2, b2):
    src = edge_index[0]
    dst = edge_index[1]
    ew = weight.astype(jnp.float32)
    x = emb[label]
    rq = (request_quantity - request_quantity.mean()) / (request_quantity.std() + 1e-06)
    req = rq[:, None] @ Wr + br
    x = jnp.concatenate([x, req], axis=-1)
    x = _gat_conv(x, src, dst, ew, W1, a_s1, a_d1, b1)
    x = jax.nn.gelu(x, approximate=False)
    x = _gat_conv(x, src, dst, ew, W2, a_s2, a_d2, b2)
    x = jax.nn.gelu(x, approximate=False)
    return x


# speedup vs baseline: 19.8578x; 19.8578x over previous
import functools

import jax
import jax.numpy as jnp
from jax import lax
from jax.experimental import pallas as pl
from jax.experimental.pallas import tpu as pltpu
from jax.experimental.pallas import tpu_sc as plsc

_N = 100000
_E = 1600000
_NSUB = 16
_NBH = 3136
_HALF = _NSUB * _NBH
_NPAD = 2 * _HALF
_EPT = _E // _NSUB
_CH = 80
_H = 16


def _mmf_body(x_ref, w_ref, a_ref, h_ref, al_ref):
    h = jnp.dot(x_ref[...], w_ref[...], preferred_element_type=jnp.float32)
    h_ref[...] = h
    al_ref[...] = jnp.dot(h, a_ref[...], preferred_element_type=jnp.float32)


def _mm_fused(x, w, a, block=1000):
    n, k = x.shape
    m = w.shape[1]
    return pl.pallas_call(
        _mmf_body,
        grid=(n // block,),
        in_specs=[
            pl.BlockSpec((block, k), lambda i: (i, 0)),
            pl.BlockSpec((k, m), lambda i: (0, 0)),
            pl.BlockSpec((m, 4), lambda i: (0, 0)),
        ],
        out_specs=[
            pl.BlockSpec((block, m), lambda i: (i, 0)),
            pl.BlockSpec((block, 4), lambda i: (i, 0)),
        ],
        out_shape=[
            jax.ShapeDtypeStruct((n, m), jnp.float32),
            jax.ShapeDtypeStruct((n, 4), jnp.float32),
        ],
    )(x, w, a)


def _gelu(v):
    return 0.5 * v * (1.0 + lax.erf(v / jnp.sqrt(2.0).astype(jnp.float32)))


def _ng_body(m0_ref, m1_ref, d0_ref, d1_ref, b_ref, o_ref):
    eps = 1e-16
    b = b_ref[...]
    o0 = m0_ref[...] / (d0_ref[...] + eps) + b[:, :_H]
    o1 = m1_ref[...] / (d1_ref[...] + eps) + b[:, _H:]
    o_ref[:, :_H] = _gelu(o0)
    o_ref[:, _H:] = _gelu(o1)


def _norm_gelu(m0, m1, d0, d1, b, block=1000):
    d0 = d0[:, None]
    d1 = d1[:, None]
    b2 = b[None, :]
    return pl.pallas_call(
        _ng_body,
        grid=(_N // block,),
        in_specs=[
            pl.BlockSpec((block, _H), lambda i: (i, 0)),
            pl.BlockSpec((block, _H), lambda i: (i, 0)),
            pl.BlockSpec((block, 1), lambda i: (i, 0)),
            pl.BlockSpec((block, 1), lambda i: (i, 0)),
            pl.BlockSpec((1, 2 * _H), lambda i: (0, 0)),
        ],
        out_specs=pl.BlockSpec((block, 2 * _H), lambda i: (i, 0)),
        out_shape=jax.ShapeDtypeStruct((_N, 2 * _H), jnp.float32),
    )(m0, m1, d0, d1, b2)


def _edge_body(src_h, dst_h, ew_h, h0, h1, as0, as1, ad0, ad1,
               m0_out, m1_out, d0_out, d1_out,
               idx_s, idx_d, idxl, ew_v, asv, adv, exv, sv, rows, msgb,
               nodebuf, denbuf, acc_msg, acc_den, sem):
    c = lax.axis_index("c")
    s = lax.axis_index("s")
    nbase = s * _NBH

    def run(h_t, as_t, ad_t, m_out, d_out):
        ebase = s * _EPT
        for p in range(2):
            half_base = p * _HALF

            def zrow(j, carry):
                nodebuf[j, :] = jnp.zeros((16,), jnp.float32)
                return carry
            lax.fori_loop(0, _NBH, zrow, 0)

            def zden(j, carry):
                denbuf[pl.ds(j * 16, 16)] = jnp.zeros((16,), jnp.float32)
                return carry
            lax.fori_loop(0, _NBH // 16, zden, 0)

            pltpu.sync_copy(nodebuf, acc_msg.at[pl.ds(nbase, _NBH)])
            pltpu.sync_copy(denbuf, acc_den.at[pl.ds(nbase, _NBH)])
            plsc.subcore_barrier()

            def chunk(ci, carry):
                base = ebase + ci * _CH
                pltpu.sync_copy(src_h.at[pl.ds(base, _CH)], idx_s)
                pltpu.sync_copy(dst_h.at[pl.ds(base, _CH)], idx_d)
                pltpu.sync_copy(ew_h.at[pl.ds(base, _CH)], ew_v)
                pltpu.async_copy(h_t.at[idx_s], rows, sem).wait()
                pltpu.async_copy(as_t.at[idx_s], asv, sem).wait()
                pltpu.async_copy(ad_t.at[idx_d], adv, sem).wait()
                for g in range(_CH // 16):
                    sl = pl.ds(g * 16, 16)
                    av = asv[sl] + adv[sl]
                    e = jnp.where(av > 0.0, av, av * 0.2)
                    ex = jnp.exp(e)
                    exv[sl] = ex
                    sv[sl] = ex * ew_v[sl]
                    d16 = idx_d[sl] - half_base
                    inh = (d16 >= 0) & (d16 < _HALF)
                    idxl[sl] = jnp.where(inh, d16, _HALF)

                def scale(g, cc):
                    b16 = g * 16
                    sg = sv[pl.ds(b16, 16)]
                    for j in range(16):
                        msgb[b16 + j, :] = rows[b16 + j, :] * sg[j]
                    return cc
                lax.fori_loop(0, _CH // 16, scale, 0)

                pltpu.sync_copy(msgb, acc_msg.at[idxl], add=True)
                pltpu.sync_copy(exv, acc_den.at[idxl], add=True)
                return carry

            lax.fori_loop(0, _EPT // _CH, chunk, 0)
            plsc.subcore_barrier()

            pltpu.sync_copy(acc_msg.at[pl.ds(nbase, _NBH)], nodebuf)
            pltpu.sync_copy(nodebuf, m_out.at[pl.ds(half_base + nbase, _NBH)])
            pltpu.sync_copy(acc_den.at[pl.ds(nbase, _NBH)], denbuf)
            pltpu.sync_copy(denbuf, d_out.at[pl.ds(half_base + nbase, _NBH)])
            plsc.subcore_barrier()

    @pl.when(c == 0)
    def _():
        run(h0, as0, ad0, m0_out, d0_out)

    @pl.when(c == 1)
    def _():
        run(h1, as1, ad1, m1_out, d1_out)


def _edge_sc(src, dst, ew, h0, h1, as0, as1, ad0, ad1):
    mesh = plsc.VectorSubcoreMesh(core_axis_name="c", subcore_axis_name="s")
    f32 = jnp.float32
    k = functools.partial(
        pl.kernel,
        mesh=mesh,
        compiler_params=pltpu.CompilerParams(use_tc_tiling_on_sc=False),
        out_type=[
            jax.ShapeDtypeStruct((_NPAD, _H), f32),
            jax.ShapeDtypeStruct((_NPAD, _H), f32),
            jax.ShapeDtypeStruct((_NPAD,), f32),
            jax.ShapeDtypeStruct((_NPAD,), f32),
        ],
        scratch_types=[
            pltpu.VMEM((_CH,), jnp.int32),
            pltpu.VMEM((_CH,), jnp.int32),
            pltpu.VMEM((_CH,), jnp.int32),
            pltpu.VMEM((_CH,), f32),
            pltpu.VMEM((_CH,), f32),
            pltpu.VMEM((_CH,), f32),
            pltpu.VMEM((_CH,), f32),
            pltpu.VMEM((_CH,), f32),
            pltpu.VMEM((_CH, _H), f32),
            pltpu.VMEM((_CH, _H), f32),
            pltpu.VMEM((_NBH, _H), f32),
            pltpu.VMEM((_NBH,), f32),
            pltpu.VMEM_SHARED((_HALF + 16, _H), f32),
            pltpu.VMEM_SHARED((_HALF + 16,), f32),
            pltpu.SemaphoreType.DMA,
        ],
    )(_edge_body)
    return k(src, dst, ew, h0, h1, as0, as1, ad0, ad1)


def _layer(x, src, dst, ew, w, a_s, a_d, b):
    a = jnp.zeros((2 * _H, 4), jnp.float32)
    a = a.at[:_H, 0].set(a_s[0]).at[_H:, 1].set(a_s[1])
    a = a.at[:_H, 2].set(a_d[0]).at[_H:, 3].set(a_d[1])
    h, al = _mm_fused(x, w, a)
    h0 = h[:, :_H]
    h1 = h[:, _H:]
    m0, m1, d0, d1 = _edge_sc(src, dst, ew,
                              h0, h1,
                              al[:, 0], al[:, 1], al[:, 2], al[:, 3])
    return _norm_gelu(m0[:_N], m1[:_N], d0[:_N], d1[:_N], b)


def kernel(label, edge_index, weight, request_quantity, emb, Wr, br,
           W1, a_s1, a_d1, b1, W2, a_s2, a_d2, b2):
    src = edge_index[0]
    dst = edge_index[1]
    ew = weight.astype(jnp.float32)
    x = emb[label]
    rq = (request_quantity - request_quantity.mean()) / (request_quantity.std() + 1e-06)
    req = rq[:, None] @ Wr + br
    x = jnp.concatenate([x, req], axis=-1)
    x = _layer(x, src, dst, ew, W1, a_s1, a_d1, b1)
    x = _layer(x, src, dst, ew, W2, a_s2, a_d2, b2)
    return x


# speedup vs baseline: 53.5427x; 2.6963x over previous
import functools

import jax
import jax.numpy as jnp
from jax import lax
from jax.experimental import pallas as pl
from jax.experimental.pallas import tpu as pltpu
from jax.experimental.pallas import tpu_sc as plsc

_N = 100000
_E = 1600000
_NSUB = 16
_NBH = 3136
_HALF = _NSUB * _NBH
_NPAD = 2 * _HALF
_CH = 128
_EPT = 100352
_EPAD = _NSUB * _EPT
_H = 16


def _mmf_body(x_ref, w_ref, a_ref, h_ref, al_ref):
    h = jnp.dot(x_ref[...], w_ref[...], preferred_element_type=jnp.float32)
    h_ref[...] = h
    al_ref[...] = jnp.dot(h, a_ref[...], preferred_element_type=jnp.float32)


def _mm_fused(x, w, a, block=1000):
    n, k = x.shape
    m = w.shape[1]
    return pl.pallas_call(
        _mmf_body,
        grid=(n // block,),
        in_specs=[
            pl.BlockSpec((block, k), lambda i: (i, 0)),
            pl.BlockSpec((k, m), lambda i: (0, 0)),
            pl.BlockSpec((m, 4), lambda i: (0, 0)),
        ],
        out_specs=[
            pl.BlockSpec((block, m), lambda i: (i, 0)),
            pl.BlockSpec((block, 4), lambda i: (i, 0)),
        ],
        out_shape=[
            jax.ShapeDtypeStruct((n, m), jnp.float32),
            jax.ShapeDtypeStruct((n, 4), jnp.float32),
        ],
    )(x, w, a)


def _gelu(v):
    return 0.5 * v * (1.0 + lax.erf(v / jnp.sqrt(2.0).astype(jnp.float32)))


def _ng_body(m0_ref, m1_ref, d0_ref, d1_ref, b_ref, o_ref):
    eps = 1e-16
    b = b_ref[...]
    o0 = m0_ref[...] / (d0_ref[...] + eps) + b[:, :_H]
    o1 = m1_ref[...] / (d1_ref[...] + eps) + b[:, _H:]
    o_ref[:, :_H] = _gelu(o0)
    o_ref[:, _H:] = _gelu(o1)


def _norm_gelu(m0, m1, d0, d1, b, block=1000):
    d0 = d0[:, None]
    d1 = d1[:, None]
    b2 = b[None, :]
    return pl.pallas_call(
        _ng_body,
        grid=(_N // block,),
        in_specs=[
            pl.BlockSpec((block, _H), lambda i: (i, 0)),
            pl.BlockSpec((block, _H), lambda i: (i, 0)),
            pl.BlockSpec((block, 1), lambda i: (i, 0)),
            pl.BlockSpec((block, 1), lambda i: (i, 0)),
            pl.BlockSpec((1, 2 * _H), lambda i: (0, 0)),
        ],
        out_specs=pl.BlockSpec((block, 2 * _H), lambda i: (i, 0)),
        out_shape=jax.ShapeDtypeStruct((_N, 2 * _H), jnp.float32),
    )(m0, m1, d0, d1, b2)


def _edge_body(src_h, dst_h, ew_h, h0, h1, as0, as1, ad0, ad1,
               m0_out, m1_out, d0_out, d1_out,
               idx_s, idx_d, idxl, ew_v, asv, adv, exv, sv, rows, msgb,
               nodebuf, denbuf, acc_msg, acc_den, sem, sem2, sem3):
    c = lax.axis_index("c")
    s = lax.axis_index("s")
    nbase = s * _NBH

    def run(h_t, as_t, ad_t, m_out, d_out):
        ebase = s * _EPT
        for p in range(2):
            half_base = p * _HALF

            def zrow(j, carry):
                nodebuf[j, :] = jnp.zeros((16,), jnp.float32)
                return carry
            lax.fori_loop(0, _NBH, zrow, 0)

            def zden(j, carry):
                denbuf[pl.ds(j * 16, 16)] = jnp.zeros((16,), jnp.float32)
                return carry
            lax.fori_loop(0, _NBH // 16, zden, 0)

            pltpu.sync_copy(nodebuf, acc_msg.at[pl.ds(nbase, _NBH)])
            pltpu.sync_copy(denbuf, acc_den.at[pl.ds(nbase, _NBH)])
            plsc.subcore_barrier()

            def chunk(ci, carry):
                base = ebase + ci * _CH
                c1 = pltpu.async_copy(src_h.at[pl.ds(base, _CH)], idx_s, sem)
                c2 = pltpu.async_copy(dst_h.at[pl.ds(base, _CH)], idx_d, sem2)
                c3 = pltpu.async_copy(ew_h.at[pl.ds(base, _CH)], ew_v, sem3)
                c1.wait()
                c2.wait()
                c3.wait()
                g1 = pltpu.async_copy(h_t.at[idx_s], rows, sem)
                g2 = pltpu.async_copy(as_t.at[idx_s], asv, sem2)
                g3 = pltpu.async_copy(ad_t.at[idx_d], adv, sem3)
                g1.wait()
                g2.wait()
                g3.wait()
                for g in range(_CH // 16):
                    sl = pl.ds(g * 16, 16)
                    av = asv[sl] + adv[sl]
                    e = jnp.where(av > 0.0, av, av * 0.2)
                    ex = jnp.exp(e)
                    exv[sl] = ex
                    sv[sl] = ex * ew_v[sl]
                    d16 = idx_d[sl] - half_base
                    inh = (d16 >= 0) & (d16 < _HALF)
                    idxl[sl] = jnp.where(inh, d16, _HALF)

                def scale(g, cc):
                    b16 = g * 16
                    sg = sv[pl.ds(b16, 16)]
                    for j in range(16):
                        msgb[b16 + j, :] = rows[b16 + j, :] * sg[j]
                    return cc
                lax.fori_loop(0, _CH // 16, scale, 0)

                s1 = pltpu.async_copy(msgb, acc_msg.at[idxl], sem, add=True)
                s2 = pltpu.async_copy(exv, acc_den.at[idxl], sem2, add=True)
                s1.wait()
                s2.wait()
                return carry

            lax.fori_loop(0, _EPT // _CH, chunk, 0)
            plsc.subcore_barrier()

            pltpu.sync_copy(acc_msg.at[pl.ds(nbase, _NBH)], nodebuf)
            pltpu.sync_copy(nodebuf, m_out.at[pl.ds(half_base + nbase, _NBH)])
            pltpu.sync_copy(acc_den.at[pl.ds(nbase, _NBH)], denbuf)
            pltpu.sync_copy(denbuf, d_out.at[pl.ds(half_base + nbase, _NBH)])
            plsc.subcore_barrier()

    @pl.when(c == 0)
    def _():
        run(h0, as0, ad0, m0_out, d0_out)

    @pl.when(c == 1)
    def _():
        run(h1, as1, ad1, m1_out, d1_out)


def _edge_sc(src, dst, ew, h0, h1, as0, as1, ad0, ad1):
    mesh = plsc.VectorSubcoreMesh(core_axis_name="c", subcore_axis_name="s")
    f32 = jnp.float32
    k = functools.partial(
        pl.kernel,
        mesh=mesh,
        compiler_params=pltpu.CompilerParams(use_tc_tiling_on_sc=False),
        out_type=[
            jax.ShapeDtypeStruct((_NPAD, _H), f32),
            jax.ShapeDtypeStruct((_NPAD, _H), f32),
            jax.ShapeDtypeStruct((_NPAD,), f32),
            jax.ShapeDtypeStruct((_NPAD,), f32),
        ],
        scratch_types=[
            pltpu.VMEM((_CH,), jnp.int32),
            pltpu.VMEM((_CH,), jnp.int32),
            pltpu.VMEM((_CH,), jnp.int32),
            pltpu.VMEM((_CH,), f32),
            pltpu.VMEM((_CH,), f32),
            pltpu.VMEM((_CH,), f32),
            pltpu.VMEM((_CH,), f32),
            pltpu.VMEM((_CH,), f32),
            pltpu.VMEM((_CH, _H), f32),
            pltpu.VMEM((_CH, _H), f32),
            pltpu.VMEM((_NBH, _H), f32),
            pltpu.VMEM((_NBH,), f32),
            pltpu.VMEM_SHARED((_HALF + 16, _H), f32),
            pltpu.VMEM_SHARED((_HALF + 16,), f32),
            pltpu.SemaphoreType.DMA,
            pltpu.SemaphoreType.DMA,
            pltpu.SemaphoreType.DMA,
        ],
    )(_edge_body)
    pad = _EPAD - _E
    src_p = jnp.concatenate([src, jnp.zeros((pad,), jnp.int32)])
    dst_p = jnp.concatenate([dst, jnp.full((pad,), _N, jnp.int32)])
    ew_p = jnp.concatenate([ew, jnp.zeros((pad,), jnp.float32)])
    return k(src_p, dst_p, ew_p, h0, h1, as0, as1, ad0, ad1)


def _layer(x, src, dst, ew, w, a_s, a_d, b):
    a = jnp.zeros((2 * _H, 4), jnp.float32)
    a = a.at[:_H, 0].set(a_s[0]).at[_H:, 1].set(a_s[1])
    a = a.at[:_H, 2].set(a_d[0]).at[_H:, 3].set(a_d[1])
    h, al = _mm_fused(x, w, a)
    h0 = h[:, :_H]
    h1 = h[:, _H:]
    m0, m1, d0, d1 = _edge_sc(src, dst, ew,
                              h0, h1,
                              al[:, 0], al[:, 1], al[:, 2], al[:, 3])
    return _norm_gelu(m0[:_N], m1[:_N], d0[:_N], d1[:_N], b)


def kernel(label, edge_index, weight, request_quantity, emb, Wr, br,
           W1, a_s1, a_d1, b1, W2, a_s2, a_d2, b2):
    src = edge_index[0]
    dst = edge_index[1]
    ew = weight.astype(jnp.float32)
    x = emb[label]
    rq = (request_quantity - request_quantity.mean()) / (request_quantity.std() + 1e-06)
    req = rq[:, None] @ Wr + br
    x = jnp.concatenate([x, req], axis=-1)
    x = _layer(x, src, dst, ew, W1, a_s1, a_d1, b1)
    x = _layer(x, src, dst, ew, W2, a_s2, a_d2, b2)
    return x
